# Initial kernel scaffold; baseline (speedup 1.0000x reference)
import jax
import jax.numpy as jnp
from jax.experimental import pallas as pl


def kernel(x_cumulative, x_windowed, edge_index, edge_attr, simple_edge_index, simple_edge_weight, Wq, bq, Wk, bk, Wv, bv, We, Wskip, bskip, nn_gamma, nn_beta, initial_weight, gru_Wih, gru_Whh, gru_bih, gru_bhh, en_gamma, en_beta, cW1, cb1, cln_g, cln_b, cW2, cb2):
    raise NotImplementedError("write your pallas kernel here")


# speedup vs baseline: 13.7457x; 13.7457x over previous
import functools

import jax
import jax.numpy as jnp
from jax import lax
from jax.experimental import pallas as pl
from jax.experimental.pallas import tpu as pltpu
from jax.experimental.pallas import tpu_sc as plsc

NC = 2
NS = 16
NW = NC * NS


def _mesh():
    return plsc.VectorSubcoreMesh(core_axis_name="c", subcore_axis_name="s",
                                  num_cores=NC, num_subcores=NS)


def _chunking(E):
    per_w = E // NW
    assert per_w * NW == E
    B = 200
    assert per_w % B == 0 and B % 8 == 0 and per_w % 8 == 0
    return per_w, B, per_w // B


def _sc_gather(tables, idxs):
    E = idxs[0].shape[0]
    per_w, B, nch = _chunking(E)
    n = len(tables)
    out_type = tuple(jax.ShapeDtypeStruct((E, t.shape[1]), jnp.float32)
                     for t in tables)
    scratch = []
    for t in tables:
        scratch.append(pltpu.VMEM((B,), jnp.int32))
        scratch.append(pltpu.VMEM((B, t.shape[1]), jnp.float32))
        scratch.append(pltpu.SemaphoreType.DMA)

    @functools.partial(pl.kernel, out_type=out_type, mesh=_mesh(),
                       scratch_types=tuple(scratch))
    def kfn(*refs):
        tab = refs[:n]
        idx = refs[n:2 * n]
        out = refs[2 * n:3 * n]
        scr = refs[3 * n:]
        wid = lax.axis_index("s") * NC + lax.axis_index("c")

        def body(c, carry):
            base = wid * per_w + c * B
            descs = []
            for j in range(n):
                ib, db, sem = scr[3 * j], scr[3 * j + 1], scr[3 * j + 2]
                pltpu.sync_copy(idx[j].at[pl.ds(base, B)], ib)
                descs.append(pltpu.async_copy(tab[j].at[ib], db, sem))
            for j in range(n):
                descs[j].wait()
                pltpu.sync_copy(scr[3 * j + 1], out[j].at[pl.ds(base, B)])
            return carry

        lax.fori_loop(0, nch, body, 0)

    res = kfn(*tables, *idxs)
    return list(res) if isinstance(res, (tuple, list)) else [res]


def _sc_scatter_add(vals, idx, nrows):
    E, Dr = vals.shape
    per_w, B, nch = _chunking(E)
    align = NS * 8
    nrp = ((nrows + align - 1) // align) * align
    rpt = nrp // NS
    zeros = jnp.zeros((rpt, Dr), jnp.float32)

    @functools.partial(
        pl.kernel,
        out_type=jax.ShapeDtypeStruct((NC, nrp, Dr), jnp.float32),
        mesh=_mesh(),
        scratch_types=(pltpu.VMEM((B,), jnp.int32),
                       pltpu.VMEM((B, Dr), jnp.float32),
                       pltpu.VMEM_SHARED((nrp, Dr), jnp.float32)),
        compiler_params=pltpu.CompilerParams(use_tc_tiling_on_sc=False))
    def kfn(vals_hbm, idx_hbm, z_hbm, out_hbm, ib, vb, acc):
        cid = lax.axis_index("c")
        sid = lax.axis_index("s")
        wid = sid * NC + cid
        pltpu.sync_copy(z_hbm, acc.at[pl.ds(sid * rpt, rpt)])
        plsc.subcore_barrier()

        def body(c, carry):
            base = wid * per_w + c * B
            pltpu.sync_copy(idx_hbm.at[pl.ds(base, B)], ib)
            pltpu.sync_copy(vals_hbm.at[pl.ds(base, B)], vb)
            pltpu.sync_copy(vb, acc.at[ib], add=True)
            return carry

        lax.fori_loop(0, nch, body, 0)
        plsc.subcore_barrier()
        pltpu.sync_copy(acc.at[pl.ds(sid * rpt, rpt)],
                        out_hbm.at[cid, pl.ds(sid * rpt, rpt)])

    return kfn(vals, idx, zeros)


def _row_spec(b, w):
    return pl.BlockSpec((b, w), lambda i: (i, 0))


def _full_spec(shape):
    nd = len(shape)
    return pl.BlockSpec(shape, lambda i: (0,) * nd)


def _tc_qkvs(x, Wq, Wk, Wv, Wskip, bq, bk, bv, bskip, BN):
    N, F = x.shape
    D = Wq.shape[1]
    grid = (N // BN,)

    def body(x_ref, wq, wk, wv, ws, b1, b2, b3, b4, q_ref, k_ref, v_ref, s_ref):
        xb = x_ref[...]
        q_ref[...] = jnp.dot(xb, wq[...], preferred_element_type=jnp.float32) + b1[...]
        k_ref[...] = jnp.dot(xb, wk[...], preferred_element_type=jnp.float32) + b2[...]
        v_ref[...] = jnp.dot(xb, wv[...], preferred_element_type=jnp.float32) + b3[...]
        s_ref[...] = jnp.dot(xb, ws[...], preferred_element_type=jnp.float32) + b4[...]

    outs = pl.pallas_call(
        body, grid=grid,
        in_specs=[_row_spec(BN, F)] + [_full_spec((F, D))] * 4 + [_full_spec((1, D))] * 4,
        out_specs=[_row_spec(BN, D)] * 4,
        out_shape=[jax.ShapeDtypeStruct((N, D), jnp.float32)] * 4,
    )(x, Wq, Wk, Wv, Wskip, bq.reshape(1, D), bk.reshape(1, D),
      bv.reshape(1, D), bskip.reshape(1, D))
    return outs


def _tc_edge_proj(edge_attr, We, BE):
    E, ED = edge_attr.shape
    D = We.shape[1]

    def body(a_ref, w_ref, e_ref):
        e_ref[...] = jnp.dot(a_ref[...], w_ref[...],
                             preferred_element_type=jnp.float32)

    return pl.pallas_call(
        body, grid=(E // BE,),
        in_specs=[_row_spec(BE, ED), _full_spec((ED, D))],
        out_specs=_row_spec(BE, D),
        out_shape=jax.ShapeDtypeStruct((E, D), jnp.float32),
    )(edge_attr, We)


def _tc_pay(qd, ks, vs, e, BE, C):
    E, D = qd.shape
    inv_sqrt_c = 1.0 / (C ** 0.5)

    def body(qd_ref, ks_ref, vs_ref, e_ref, p_ref):
        q = qd_ref[...]
        k = ks_ref[...] + e_ref[...]
        a0 = jnp.sum(q[:, :C] * k[:, :C], axis=1, keepdims=True) * inv_sqrt_c
        a1 = jnp.sum(q[:, C:] * k[:, C:], axis=1, keepdims=True) * inv_sqrt_c
        ex0 = jnp.exp(a0)
        ex1 = jnp.exp(a1)
        ve = vs_ref[...] + e_ref[...]
        p_ref[...] = jnp.concatenate(
            [ex0 * ve[:, :C], ex1 * ve[:, C:], ex0, ex1,
             jnp.zeros((ve.shape[0], 14), jnp.float32)], axis=1)

    return pl.pallas_call(
        body, grid=(E // BE,),
        in_specs=[_row_spec(BE, D)] * 4,
        out_specs=_row_spec(BE, D + 16),
        out_shape=jax.ShapeDtypeStruct((E, D + 16), jnp.float32),
    )(qd, ks, vs, e)


def _layer_norm(x, g, b, eps=1e-5):
    m = jnp.mean(x, axis=1, keepdims=True)
    v = jnp.mean((x - m) ** 2, axis=1, keepdims=True)
    return (x - m) * jax.lax.rsqrt(v + eps) * g + b


def _part_specs(BN, W):
    return [pl.BlockSpec((1, BN, W), lambda i: (0, i, 0)),
            pl.BlockSpec((1, BN, W), lambda i: (1, i, 0))]


def _tc_h(acc, xskip, gamma, beta, BN, C):
    N = xskip.shape[0]
    D = 2 * C
    W = acc.shape[2]

    def body(a0_ref, a1_ref, sk_ref, g_ref, b_ref, h_ref):
        a = a0_ref[0] + a1_ref[0]
        num = a[:, :D]
        den0 = a[:, D:D + 1]
        den1 = a[:, D + 1:D + 2]
        div = jnp.concatenate([jnp.broadcast_to(den0, (num.shape[0], C)),
                               jnp.broadcast_to(den1, (num.shape[0], C))], axis=1)
        out = jnp.where(div > 0.0, num / jnp.maximum(div, 1e-30), 0.0)
        out = out + sk_ref[...]
        h_ref[...] = jnp.maximum(_layer_norm(out, g_ref[...], b_ref[...]), 0.0)

    return pl.pallas_call(
        body, grid=(N // BN,),
        in_specs=_part_specs(BN, W) + [_row_spec(BN, D),
                  _full_spec((1, D)), _full_spec((1, D))],
        out_specs=_row_spec(BN, D),
        out_shape=jax.ShapeDtypeStruct((N, D), jnp.float32),
    )(acc, acc, xskip, gamma.reshape(1, D), beta.reshape(1, D))


def _sigmoid(x):
    return 1.0 / (1.0 + jnp.exp(-x))


def _tc_gru(xw, Wih, Whh, bih, bhh):
    D = xw.shape[0]

    def body(x_ref, wih_ref, whh_ref, bi_ref, bh_ref, w_ref):
        x = x_ref[...]
        gi = lax.dot_general(x, wih_ref[...], (((1,), (1,)), ((), ())),
                             preferred_element_type=jnp.float32) + bi_ref[...]
        gh = lax.dot_general(x, whh_ref[...], (((1,), (1,)), ((), ())),
                             preferred_element_type=jnp.float32) + bh_ref[...]
        i_r, i_z, i_n = gi[:, :D], gi[:, D:2 * D], gi[:, 2 * D:]
        h_r, h_z, h_n = gh[:, :D], gh[:, D:2 * D], gh[:, 2 * D:]
        r = _sigmoid(i_r + h_r)
        z = _sigmoid(i_z + h_z)
        ng = jnp.tanh(i_n + r * h_n)
        w_ref[...] = (1.0 - z) * ng + z * x

    return pl.pallas_call(
        body,
        in_specs=[pl.BlockSpec((D, D), lambda: (0, 0)),
                  pl.BlockSpec((3 * D, D), lambda: (0, 0)),
                  pl.BlockSpec((3 * D, D), lambda: (0, 0)),
                  pl.BlockSpec((1, 3 * D), lambda: (0, 0)),
                  pl.BlockSpec((1, 3 * D), lambda: (0, 0))],
        out_specs=pl.BlockSpec((D, D), lambda: (0, 0)),
        out_shape=jax.ShapeDtypeStruct((D, D), jnp.float32),
    )(xw, Wih, Whh, bih.reshape(1, 3 * D), bhh.reshape(1, 3 * D))


def _tc_g(h, W, degp, BN):
    N, D = h.shape
    WD = degp.shape[2]

    def body(h_ref, w_ref, d0_ref, d1_ref, g_ref):
        deg = d0_ref[0][:, :1] + d1_ref[0][:, :1] + 1.0
        dinv = jax.lax.rsqrt(deg)
        hw = jnp.dot(h_ref[...], w_ref[...], preferred_element_type=jnp.float32)
        g_ref[...] = hw * dinv

    return pl.pallas_call(
        body, grid=(N // BN,),
        in_specs=[_row_spec(BN, D), _full_spec((D, D))] + _part_specs(BN, WD),
        out_specs=_row_spec(BN, D),
        out_shape=jax.ShapeDtypeStruct((N, D), jnp.float32),
    )(h, W, degp, degp)


def _tc_scale(rows, w, BE):
    E, D = rows.shape

    def body(r_ref, w_ref, o_ref):
        o_ref[...] = r_ref[...] * w_ref[...]

    return pl.pallas_call(
        body, grid=(E // BE,),
        in_specs=[_row_spec(BE, D), _row_spec(BE, 1)],
        out_specs=_row_spec(BE, D),
        out_shape=jax.ShapeDtypeStruct((E, D), jnp.float32),
    )(rows, w.reshape(E, 1))


def _tc_out2_pool(scatp, g, degp, gamma, beta, BN):
    N, D = g.shape
    WD = degp.shape[2]

    def body(s0_ref, s1_ref, g_ref, d0_ref, d1_ref, ga_ref, be_ref,
             sum_ref, max_ref):
        i = pl.program_id(0)
        deg = d0_ref[0][:, :1] + d1_ref[0][:, :1] + 1.0
        dinv = jax.lax.rsqrt(deg)
        out2 = dinv * (s0_ref[0] + s1_ref[0] + g_ref[...])
        h2 = jnp.maximum(_layer_norm(out2, ga_ref[...], be_ref[...]), 0.0)
        bsum = jnp.sum(h2, axis=0, keepdims=True)
        bmax = jnp.max(h2, axis=0, keepdims=True)

        @pl.when(i == 0)
        def _():
            sum_ref[...] = bsum
            max_ref[...] = bmax

        @pl.when(i > 0)
        def _():
            sum_ref[...] = sum_ref[...] + bsum
            max_ref[...] = jnp.maximum(max_ref[...], bmax)

    return pl.pallas_call(
        body, grid=(N // BN,),
        in_specs=_part_specs(BN, D) + [_row_spec(BN, D)] + _part_specs(BN, WD)
                 + [_full_spec((1, D)), _full_spec((1, D))],
        out_specs=[pl.BlockSpec((1, D), lambda i: (0, 0))] * 2,
        out_shape=[jax.ShapeDtypeStruct((1, D), jnp.float32)] * 2,
    )(scatp, scatp, g, degp, degp, gamma.reshape(1, D), beta.reshape(1, D))


def _tc_classifier(psum, pmax, n_nodes, cW1, cb1, cln_g, cln_b, cW2, cb2):
    D = psum.shape[1]
    K = cW1.shape[1]

    def body(ps_ref, pm_ref, w1_ref, b1_ref, g_ref, be_ref, w2_ref, b2_ref,
             o_ref):
        pooled = jnp.concatenate([ps_ref[...] * (1.0 / n_nodes), pm_ref[...]],
                                 axis=1)
        z1 = jnp.dot(pooled, w1_ref[...], preferred_element_type=jnp.float32) + b1_ref[...]
        z1 = jnp.maximum(_layer_norm(z1, g_ref[...], be_ref[...]), 0.0)
        o_ref[...] = jnp.dot(z1, w2_ref[...], preferred_element_type=jnp.float32) + b2_ref[...]

    return pl.pallas_call(
        body,
        in_specs=[pl.BlockSpec((1, D), lambda: (0, 0)),
                  pl.BlockSpec((1, D), lambda: (0, 0)),
                  pl.BlockSpec((2 * D, K), lambda: (0, 0)),
                  pl.BlockSpec((1, K), lambda: (0, 0)),
                  pl.BlockSpec((1, K), lambda: (0, 0)),
                  pl.BlockSpec((1, K), lambda: (0, 0)),
                  pl.BlockSpec((K, 1), lambda: (0, 0)),
                  pl.BlockSpec((1, 1), lambda: (0, 0))],
        out_specs=pl.BlockSpec((1, 1), lambda: (0, 0)),
        out_shape=jax.ShapeDtypeStruct((1, 1), jnp.float32),
    )(psum, pmax, cW1, cb1.reshape(1, K), cln_g.reshape(1, K),
      cln_b.reshape(1, K), cW2, cb2.reshape(1, 1))


def kernel(x_cumulative, x_windowed, edge_index, edge_attr, simple_edge_index,
           simple_edge_weight, Wq, bq, Wk, bk, Wv, bv, We, Wskip, bskip,
           nn_gamma, nn_beta, initial_weight, gru_Wih, gru_Whh, gru_bih,
           gru_bhh, en_gamma, en_beta, cW1, cb1, cln_g, cln_b, cW2, cb2):
    N = x_cumulative.shape[0]
    E = edge_index.shape[1]
    D = Wq.shape[1]
    C = D // 2
    BN, BE = 1000, 2000

    x = jnp.concatenate([x_cumulative, x_windowed], axis=1)
    src, dst = edge_index[0], edge_index[1]
    ssrc, sdst = simple_edge_index[0], simple_edge_index[1]

    q, k, v, xskip = _tc_qkvs(x, Wq, Wk, Wv, Wskip, bq, bk, bv, bskip, BN)
    e = _tc_edge_proj(edge_attr, We, BE)

    ks, qd, vs = _sc_gather([k, q, v], [src, dst, src])

    w16 = jnp.pad(simple_edge_weight[:, None], ((0, 0), (0, 15)))
    degp = _sc_scatter_add(w16, sdst, N)

    pay = _tc_pay(qd, ks, vs, e, BE, C)

    accp = _sc_scatter_add(pay, dst, N)

    h = _tc_h(accp, xskip, nn_gamma, nn_beta, BN, C)
    Wg = _tc_gru(initial_weight[0], gru_Wih, gru_Whh, gru_bih, gru_bhh)
    g = _tc_g(h, Wg, degp, BN)

    (grows,) = _sc_gather([g], [ssrc])
    scaled = _tc_scale(grows, simple_edge_weight, BE)
    scatp = _sc_scatter_add(scaled, sdst, N)

    psum, pmax = _tc_out2_pool(scatp, g, degp, en_gamma, en_beta, BN)
    return _tc_classifier(psum, pmax, float(N), cW1, cb1, cln_g, cln_b,
                          cW2, cb2)
